# baseline (speedup 1.0000x reference)
import functools

import jax
import jax.numpy as jnp
from jax import lax
from jax.experimental import pallas as pl
from jax.experimental.pallas import tpu as pltpu
from jax.experimental.pallas import tpu_sc as plsc

B = 16384
L = 50
D = 32
NW = 32
U = B // NW
UCHUNK = 2
ROWS = UCHUNK * L
NCHUNK = U // UCHUNK
NBUF = 2
SCORES = U * L

_info = plsc.get_sparse_core_info()
_NC = _info.num_cores

_CBLK = 2048
_NBLK = (1000000 + 4 * _CBLK - 1) // (4 * _CBLK)
_TROWS = _NBLK * _CBLK * 4


def _linearize_body(x_ref, o_ref):
    eye = jnp.eye(D, dtype=jnp.float32)
    for q in range(4):
        xq = x_ref[:, q * _CBLK:(q + 1) * _CBLK]
        o_ref[:, q * D:(q + 1) * D] = jax.lax.dot_general(
            xq, eye, (((0,), (0,)), ((), ())),
            preferred_element_type=jnp.float32)


_linearize = pl.pallas_call(
    _linearize_body,
    grid=(_NBLK,),
    in_specs=[pl.BlockSpec((D, 4 * _CBLK), lambda i: (0, i))],
    out_specs=pl.BlockSpec((_CBLK, 128), lambda i: (i, 0)),
    out_shape=jax.ShapeDtypeStruct((_NBLK * _CBLK, 128), jnp.float32),
)


def _row_map(r):
    return ((r >> 13) << 13) + ((r & 2047) << 2) + ((r >> 11) & 3)


@functools.partial(
    pl.kernel,
    mesh=plsc.VectorSubcoreMesh(core_axis_name="c", subcore_axis_name="s"),
    out_type=jax.ShapeDtypeStruct((NW, SCORES), jnp.float32),
    scratch_types=[
        pltpu.VMEM((NCHUNK, ROWS), jnp.int32),
        pltpu.VMEM((U, D), jnp.float32),
        pltpu.VMEM((NBUF, ROWS, D), jnp.float32),
        pltpu.VMEM((SCORES + 16,), jnp.float32),
        pltpu.SemaphoreType.DMA,
        pltpu.SemaphoreType.DMA,
        pltpu.SemaphoreType.DMA,
    ],
    compiler_params=pltpu.CompilerParams(
        needs_layout_passes=False, use_tc_tiling_on_sc=False),
)
def _sc_scores(uemb_h, items_h, iw_h, out_h,
               iidx_v, urows_v, rows_v, scores_v,
               usem, isem0, isem1):
    w = lax.axis_index("s") * _NC + lax.axis_index("c")
    lane15 = lax.iota(jnp.int32, 16) == 15

    uc = pltpu.async_copy(uemb_h.at[w], urows_v, usem)
    pltpu.sync_copy(items_h.at[w], iidx_v)
    uc.wait()

    isems = (isem0, isem1)

    def issue(c, b):
        pltpu.async_copy(iw_h.at[iidx_v.at[c]], rows_v.at[b], isems[b])

    for b in range(NBUF):
        issue(b, b)

    def group(g, carry):
        for b in range(NBUF):
            c = g * NBUF + b
            pltpu.make_async_copy(
                iw_h.at[iidx_v.at[c]], rows_v.at[b], isems[b]).wait()
            rows = rows_v.at[b]
            for ul in range(UCHUNK):
                uu = c * UCHUNK + ul
                urow = urows_v.at[uu]
                u_lo = urow[pl.ds(0, 16)]
                u_hi = urow[pl.ds(16, 16)]
                sbase = c * ROWS + ul * L

                @plsc.parallel_loop(0, L, unroll=10)
                def _scores(l):
                    r = rows.at[ul * L + l]
                    q = u_lo * r[pl.ds(0, 16)] + u_hi * r[pl.ds(16, 16)]
                    cum = plsc.cumsum(q)
                    plsc.store_compressed(
                        scores_v.at[pl.ds(sbase + l, 16)], cum, mask=lane15)

            nxt = c + NBUF

            @pl.when(nxt < NCHUNK)
            def _():
                issue(nxt, b)
        return carry

    lax.fori_loop(0, NCHUNK // NBUF, group, 0)
    pltpu.sync_copy(scores_v.at[pl.ds(0, SCORES)], out_h.at[w])


def kernel(user, items, embed_user_w, embed_item_w):
    uemb = jnp.take(embed_user_w, user, axis=0, mode="clip")
    uemb3 = uemb.reshape(NW, U, D)
    items3 = _row_map(items.astype(jnp.int32)).reshape(NW, NCHUNK, ROWS)
    itab = _linearize(embed_item_w.T).reshape(_TROWS, D)
    out = _sc_scores(uemb3, items3, itab)
    return out.reshape(B, L)


# speedup vs baseline: 1.0121x; 1.0121x over previous
import functools

import jax
import jax.numpy as jnp
from jax import lax
from jax.experimental import pallas as pl
from jax.experimental.pallas import tpu as pltpu
from jax.experimental.pallas import tpu_sc as plsc

B = 16384
L = 50
D = 32
NW = 32
U = B // NW
UCHUNK = 2
ROWS = UCHUNK * L
NCHUNK = U // UCHUNK
NBUF = 2
SCORES = U * L

_info = plsc.get_sparse_core_info()
_NC = _info.num_cores

_CBLK = 4096
_NBLK = (1000000 + 4 * _CBLK - 1) // (4 * _CBLK)
_TROWS = _NBLK * _CBLK * 4


def _linearize_body(x_ref, o_ref):
    for q in range(4):
        o_ref[:, q * D:(q + 1) * D] = jnp.transpose(
            x_ref[:, q * _CBLK:(q + 1) * _CBLK])


_linearize = pl.pallas_call(
    _linearize_body,
    grid=(_NBLK,),
    in_specs=[pl.BlockSpec((D, 4 * _CBLK), lambda i: (0, i))],
    out_specs=pl.BlockSpec((_CBLK, 128), lambda i: (i, 0)),
    out_shape=jax.ShapeDtypeStruct((_NBLK * _CBLK, 128), jnp.float32),
)


_SH = _CBLK.bit_length() - 1


def _row_map(r):
    return (((r >> (_SH + 2)) << (_SH + 2))
            + ((r & (_CBLK - 1)) << 2) + ((r >> _SH) & 3))


@functools.partial(
    pl.kernel,
    mesh=plsc.VectorSubcoreMesh(core_axis_name="c", subcore_axis_name="s"),
    out_type=jax.ShapeDtypeStruct((NW, SCORES), jnp.float32),
    scratch_types=[
        pltpu.VMEM((NCHUNK, ROWS), jnp.int32),
        pltpu.VMEM((U, D), jnp.float32),
        pltpu.VMEM((NBUF, ROWS, D), jnp.float32),
        pltpu.VMEM((SCORES + 16,), jnp.float32),
        pltpu.SemaphoreType.DMA,
        pltpu.SemaphoreType.DMA,
        pltpu.SemaphoreType.DMA,
    ],
    compiler_params=pltpu.CompilerParams(
        needs_layout_passes=False, use_tc_tiling_on_sc=False),
)
def _sc_scores(uemb_h, items_h, iw_h, out_h,
               iidx_v, urows_v, rows_v, scores_v,
               usem, isem0, isem1):
    w = lax.axis_index("s") * _NC + lax.axis_index("c")
    lane15 = lax.iota(jnp.int32, 16) == 15

    uc = pltpu.async_copy(uemb_h.at[w], urows_v, usem)
    pltpu.sync_copy(items_h.at[w], iidx_v)
    uc.wait()

    isems = (isem0, isem1)

    def issue(c, b):
        pltpu.async_copy(iw_h.at[iidx_v.at[c]], rows_v.at[b], isems[b])

    for b in range(NBUF):
        issue(b, b)

    def group(g, carry):
        for b in range(NBUF):
            c = g * NBUF + b
            pltpu.make_async_copy(
                iw_h.at[iidx_v.at[c]], rows_v.at[b], isems[b]).wait()
            rows = rows_v.at[b]
            for ul in range(UCHUNK):
                uu = c * UCHUNK + ul
                urow = urows_v.at[uu]
                u_lo = urow[pl.ds(0, 16)]
                u_hi = urow[pl.ds(16, 16)]
                sbase = c * ROWS + ul * L

                @plsc.parallel_loop(0, L, unroll=10)
                def _scores(l):
                    r = rows.at[ul * L + l]
                    q = u_lo * r[pl.ds(0, 16)] + u_hi * r[pl.ds(16, 16)]
                    cum = plsc.cumsum(q)
                    plsc.store_compressed(
                        scores_v.at[pl.ds(sbase + l, 16)], cum, mask=lane15)

            nxt = c + NBUF

            @pl.when(nxt < NCHUNK)
            def _():
                issue(nxt, b)
        return carry

    lax.fori_loop(0, NCHUNK // NBUF, group, 0)
    pltpu.sync_copy(scores_v.at[pl.ds(0, SCORES)], out_h.at[w])


def kernel(user, items, embed_user_w, embed_item_w):
    uemb = jnp.take(embed_user_w, user, axis=0, mode="clip")
    uemb3 = uemb.reshape(NW, U, D)
    items3 = _row_map(items.astype(jnp.int32)).reshape(NW, NCHUNK, ROWS)
    itab = _linearize(embed_item_w.T).reshape(_TROWS, D)
    out = _sc_scores(uemb3, items3, itab)
    return out.reshape(B, L)
